# Initial kernel scaffold; baseline (speedup 1.0000x reference)
import functools

import jax
import jax.numpy as jnp
from jax import lax
from jax.experimental import pallas as pl
from jax.experimental.pallas import tpu as pltpu
from jax.experimental.pallas import tpu_sc as plsc

N = 50000
E = 800000
G = 512
HID = 64
NP = 53248
ER = 6400
BR = 416
SR = NP // 16
GP = 520

_mesh = plsc.VectorSubcoreMesh(core_axis_name="c", subcore_axis_name="s")


@functools.partial(
    pl.kernel,
    out_type=jax.ShapeDtypeStruct((2, NP, 32), jnp.float32),
    mesh=_mesh,
    scratch_types=[
        pltpu.VMEM((8, 128), jnp.int32),
        pltpu.VMEM((8, 128), jnp.int32),
        pltpu.VMEM((1024, 32), jnp.float32),
        pltpu.VMEM_SHARED((NP, 32), jnp.float32),
        pltpu.SemaphoreType.DMA,
    ],
)
def _sc_agg64(h_hbm, src_hbm, dst_hbm, zeros_hbm, out_hbm,
              sidx, didx, rows, acc, gsem):
    c = lax.axis_index("c")
    s = lax.axis_index("s")
    pltpu.sync_copy(zeros_hbm, acc.at[pl.ds(s * SR, SR)])
    plsc.subcore_barrier()

    rows_per_tile = ER // 16

    def chunk(i, _):
        base = s * rows_per_tile + i * 8
        pltpu.sync_copy(src_hbm.at[pl.ds(base, 8)], sidx)
        pltpu.sync_copy(dst_hbm.at[pl.ds(base, 8)], didx)
        cps = [
            pltpu.async_copy(h_hbm.at[c].at[sidx.at[j]],
                             rows.at[pl.ds(j * 128, 128)], gsem)
            for j in range(8)
        ]
        for cp in cps:
            cp.wait()
        for j in range(8):
            pltpu.sync_copy(rows.at[pl.ds(j * 128, 128)],
                            acc.at[didx.at[j]], add=True)
        return 0

    lax.fori_loop(0, rows_per_tile // 8, chunk, 0, unroll=False)
    plsc.subcore_barrier()
    pltpu.sync_copy(acc.at[pl.ds(s * SR, SR)],
                    out_hbm.at[c, pl.ds(s * SR, SR)])


@functools.partial(
    pl.kernel,
    out_type=jax.ShapeDtypeStruct((2, NP, 8), jnp.float32),
    mesh=_mesh,
    scratch_types=[
        pltpu.VMEM((8, 128), jnp.int32),
        pltpu.VMEM((8, 128), jnp.int32),
        pltpu.VMEM((1024, 8), jnp.float32),
        pltpu.VMEM_SHARED((NP, 8), jnp.float32),
        pltpu.SemaphoreType.DMA,
    ],
)
def _sc_agg8(x_hbm, src_hbm, dst_hbm, zeros_hbm, out_hbm,
             sidx, didx, rows, acc, gsem):
    c = lax.axis_index("c")
    s = lax.axis_index("s")
    pltpu.sync_copy(zeros_hbm, acc.at[pl.ds(s * SR, SR)])
    plsc.subcore_barrier()

    rows_per_tile = ER // 32

    def chunk(i, _):
        base = (c * 16 + s) * rows_per_tile + i * 8
        pltpu.sync_copy(src_hbm.at[pl.ds(base, 8)], sidx)
        pltpu.sync_copy(dst_hbm.at[pl.ds(base, 8)], didx)
        cps = [
            pltpu.async_copy(x_hbm.at[sidx.at[j]],
                             rows.at[pl.ds(j * 128, 128)], gsem)
            for j in range(8)
        ]
        for cp in cps:
            cp.wait()
        for j in range(8):
            pltpu.sync_copy(rows.at[pl.ds(j * 128, 128)],
                            acc.at[didx.at[j]], add=True)
        return 0

    lax.fori_loop(0, rows_per_tile // 8, chunk, 0, unroll=False)
    plsc.subcore_barrier()
    pltpu.sync_copy(acc.at[pl.ds(s * SR, SR)],
                    out_hbm.at[c, pl.ds(s * SR, SR)])


@functools.partial(
    pl.kernel,
    out_type=[jax.ShapeDtypeStruct((4, 2, G, 32), jnp.float32),
              jax.ShapeDtypeStruct((G, 8), jnp.float32)],
    mesh=_mesh,
    scratch_types=[
        pltpu.VMEM((26, 128), jnp.int32),
        pltpu.VMEM((128, 32), jnp.float32),
        pltpu.VMEM((128, 8), jnp.float32),
        pltpu.VMEM_SHARED((GP, 32), jnp.float32),
        pltpu.VMEM_SHARED((GP, 32), jnp.float32),
        pltpu.VMEM_SHARED((GP, 32), jnp.float32),
        pltpu.VMEM_SHARED((GP, 32), jnp.float32),
        pltpu.VMEM_SHARED((GP, 8), jnp.float32),
    ],
)
def _sc_pool(h1, h2, h3, h4, batch_hbm, ones_hbm, z32_hbm, z8_hbm,
             out_pool, out_cnt, bidx, hrow, onesv, a1, a2, a3, a4, ac):
    c = lax.axis_index("c")
    s = lax.axis_index("s")
    accs = [a1, a2, a3, a4]
    hs = [h1, h2, h3, h4]

    @pl.when(s == 0)
    def _init():
        for a in accs:
            pltpu.sync_copy(z32_hbm, a)
        pltpu.sync_copy(z8_hbm, ac)

    pltpu.sync_copy(ones_hbm, onesv)
    pltpu.sync_copy(batch_hbm.at[pl.ds(s * 26, 26)], bidx)
    plsc.subcore_barrier()

    def body(j, _):
        node_base = (s * 26 + j) * 128
        for h, a in zip(hs, accs):
            pltpu.sync_copy(h.at[c, pl.ds(node_base, 128)], hrow)
            pltpu.sync_copy(hrow, a.at[bidx.at[j]], add=True)

        @pl.when(c == 0)
        def _cnt():
            pltpu.sync_copy(onesv, ac.at[bidx.at[j]], add=True)

        return 0

    lax.fori_loop(0, 26, body, 0, unroll=False)
    plsc.subcore_barrier()

    for t, a in enumerate(accs):
        @pl.when(s == t)
        def _wb(a=a, t=t):
            pltpu.sync_copy(a.at[pl.ds(0, G)], out_pool.at[t, c])

    @pl.when((c == 0) & (s == 4))
    def _wbc():
        pltpu.sync_copy(ac.at[pl.ds(0, G)], out_cnt)


def _bn(z, g, b):
    mask = (lax.broadcasted_iota(jnp.int32, (NP, 1), 0) < N).astype(jnp.float32)
    m = jnp.sum(z * mask, axis=0, keepdims=True) / N
    d = (z - m) * mask
    v = jnp.sum(d * d, axis=0, keepdims=True) / N
    return (z - m) / jnp.sqrt(v + 1e-5) * g + b


def _gelu(z):
    return jax.nn.gelu(z, approximate=False)


def _tc_layer_mid_body(h_ref, agg_ref, eps_ref, w1_ref, b1_ref, g1_ref,
                       bb1_ref, w2_ref, b2_ref, g2_ref, bb2_ref, out_ref):
    h = jnp.concatenate([h_ref[0], h_ref[1]], axis=-1)
    agg = jnp.concatenate([agg_ref[0], agg_ref[1]], axis=-1)
    u = (1.0 + eps_ref[0, 0]) * h + agg
    z = jnp.dot(u, w1_ref[...], preferred_element_type=jnp.float32) + b1_ref[...]
    z = _gelu(_bn(z, g1_ref[...], bb1_ref[...]))
    z = jnp.dot(z, w2_ref[...], preferred_element_type=jnp.float32) + b2_ref[...]
    hn = jnp.maximum(_bn(z, g2_ref[...], bb2_ref[...]), 0.0)
    out_ref[0] = hn[:, :32]
    out_ref[1] = hn[:, 32:]


def _tc_layer0_body(x_ref, agg_ref, eps_ref, w1_ref, b1_ref, g1_ref,
                    bb1_ref, w2_ref, b2_ref, g2_ref, bb2_ref, out_ref):
    agg = agg_ref[0] + agg_ref[1]
    u = (1.0 + eps_ref[0, 0]) * x_ref[...] + agg
    z = jnp.dot(u, w1_ref[...], preferred_element_type=jnp.float32) + b1_ref[...]
    z = _gelu(_bn(z, g1_ref[...], bb1_ref[...]))
    z = jnp.dot(z, w2_ref[...], preferred_element_type=jnp.float32) + b2_ref[...]
    hn = jnp.maximum(_bn(z, g2_ref[...], bb2_ref[...]), 0.0)
    out_ref[0] = hn[:, :32]
    out_ref[1] = hn[:, 32:]


def _tc_head_body(pool_ref, cnt_ref, w1_ref, b1_ref, w2_ref, b2_ref, out_ref):
    cnt = jnp.maximum(cnt_ref[:, 0:1], 1.0)
    parts = [pool_ref[l, c] for l in range(4) for c in range(2)]
    emb = jnp.concatenate(parts, axis=-1) / cnt
    z = _gelu(jnp.dot(emb, w1_ref[...], preferred_element_type=jnp.float32)
              + b1_ref[...])
    out_ref[...] = (jnp.dot(z, w2_ref[...], preferred_element_type=jnp.float32)
                    + b2_ref[...])


_tc_layer_mid = pl.pallas_call(
    _tc_layer_mid_body,
    out_shape=jax.ShapeDtypeStruct((2, NP, 32), jnp.float32),
)

_tc_layer0 = pl.pallas_call(
    _tc_layer0_body,
    out_shape=jax.ShapeDtypeStruct((2, NP, 32), jnp.float32),
)

_tc_head = pl.pallas_call(
    _tc_head_body,
    out_shape=jax.ShapeDtypeStruct((G, 1), jnp.float32),
)


def kernel(x, edge_index, batch, params):
    f32 = jnp.float32
    x8 = jnp.zeros((NP, 8), f32).at[:N, :6].set(x)
    src = jnp.zeros((ER * 128,), jnp.int32).at[:E].set(edge_index[0])
    src = src.reshape(ER, 128)
    dst = jnp.full((ER * 128,), N, jnp.int32).at[:E].set(edge_index[1])
    dst = dst.reshape(ER, 128)
    batchp = jnp.full((BR * 128,), G, jnp.int32).at[:N].set(batch)
    batchp = batchp.reshape(BR, 128)

    z32 = jnp.zeros((SR, 32), f32)
    z8 = jnp.zeros((SR, 8), f32)
    zg32 = jnp.zeros((GP, 32), f32)
    zg8 = jnp.zeros((GP, 8), f32)
    ones = jnp.ones((128, 8), f32)

    agg0 = _sc_agg8(x8, src, dst, z8)
    w1_0 = jnp.zeros((8, HID), f32).at[:6].set(params['c0_W1'])
    h = _tc_layer0(
        x8, agg0, params['c0_eps'].reshape(1, 1), w1_0,
        params['c0_b1'].reshape(1, HID), params['c0_bn1_g'].reshape(1, HID),
        params['c0_bn1_b'].reshape(1, HID), params['c0_W2'],
        params['c0_b2'].reshape(1, HID), params['bn0_g'].reshape(1, HID),
        params['bn0_b'].reshape(1, HID))
    hs = [h]
    for i in range(1, 4):
        agg = _sc_agg64(hs[-1], src, dst, z32)
        h = _tc_layer_mid(
            hs[-1], agg, params[f'c{i}_eps'].reshape(1, 1), params[f'c{i}_W1'],
            params[f'c{i}_b1'].reshape(1, HID),
            params[f'c{i}_bn1_g'].reshape(1, HID),
            params[f'c{i}_bn1_b'].reshape(1, HID), params[f'c{i}_W2'],
            params[f'c{i}_b2'].reshape(1, HID),
            params[f'bn{i}_g'].reshape(1, HID),
            params[f'bn{i}_b'].reshape(1, HID))
        hs.append(h)

    pooled, cnt = _sc_pool(hs[0], hs[1], hs[2], hs[3], batchp, ones, zg32, zg8)
    out = _tc_head(pooled, cnt, params['h_W1'],
                   params['h_b1'].reshape(1, HID), params['h_W2'],
                   params['h_b2'].reshape(1, 1))
    return out[:, 0]


# speedup vs baseline: 3.8273x; 3.8273x over previous
import functools

import jax
import jax.numpy as jnp
from jax import lax
from jax.experimental import pallas as pl
from jax.experimental.pallas import tpu as pltpu
from jax.experimental.pallas import tpu_sc as plsc

N = 50000
E = 800000
G = 512
HID = 64
NP = 51200
ER = 6400
BR = 400
SR = NP // 16
GP = 520

_mesh = plsc.VectorSubcoreMesh(core_axis_name="c", subcore_axis_name="s")


@functools.partial(
    pl.kernel,
    out_type=jax.ShapeDtypeStruct((2, NP, 32), jnp.float32),
    mesh=_mesh,
    scratch_types=[
        pltpu.VMEM((2, 128), jnp.int32),
        pltpu.VMEM((2, 128), jnp.int32),
        pltpu.VMEM((256, 32), jnp.float32),
        pltpu.VMEM_SHARED((NP, 32), jnp.float32),
        pltpu.SemaphoreType.DMA,
    ],
    compiler_params=pltpu.CompilerParams(use_tc_tiling_on_sc=False),
)
def _sc_agg64(h_hbm, src_hbm, dst_hbm, zeros_hbm, out_hbm,
              sidx, didx, rows, acc, gsem):
    c = lax.axis_index("c")
    s = lax.axis_index("s")
    pltpu.sync_copy(zeros_hbm, acc.at[pl.ds(s * SR, SR)])
    plsc.subcore_barrier()

    rows_per_tile = ER // 16

    def chunk(i, _):
        base = s * rows_per_tile + i * 2
        pltpu.sync_copy(src_hbm.at[pl.ds(base, 2)], sidx)
        pltpu.sync_copy(dst_hbm.at[pl.ds(base, 2)], didx)
        cps = [
            pltpu.async_copy(h_hbm.at[c].at[sidx.at[j]],
                             rows.at[pl.ds(j * 128, 128)], gsem)
            for j in range(2)
        ]
        for cp in cps:
            cp.wait()
        for j in range(2):
            pltpu.sync_copy(rows.at[pl.ds(j * 128, 128)],
                            acc.at[didx.at[j]], add=True)
        return 0

    lax.fori_loop(0, rows_per_tile // 2, chunk, 0, unroll=False)
    plsc.subcore_barrier()
    pltpu.sync_copy(acc.at[pl.ds(s * SR, SR)],
                    out_hbm.at[c, pl.ds(s * SR, SR)])


@functools.partial(
    pl.kernel,
    out_type=jax.ShapeDtypeStruct((2, NP, 8), jnp.float32),
    mesh=_mesh,
    scratch_types=[
        pltpu.VMEM((2, 128), jnp.int32),
        pltpu.VMEM((2, 128), jnp.int32),
        pltpu.VMEM((256, 8), jnp.float32),
        pltpu.VMEM_SHARED((NP, 8), jnp.float32),
        pltpu.SemaphoreType.DMA,
    ],
    compiler_params=pltpu.CompilerParams(use_tc_tiling_on_sc=False),
)
def _sc_agg8(x_hbm, src_hbm, dst_hbm, zeros_hbm, out_hbm,
             sidx, didx, rows, acc, gsem):
    c = lax.axis_index("c")
    s = lax.axis_index("s")
    pltpu.sync_copy(zeros_hbm, acc.at[pl.ds(s * SR, SR)])
    plsc.subcore_barrier()

    rows_per_tile = ER // 32

    def chunk(i, _):
        base = (c * 16 + s) * rows_per_tile + i * 2
        pltpu.sync_copy(src_hbm.at[pl.ds(base, 2)], sidx)
        pltpu.sync_copy(dst_hbm.at[pl.ds(base, 2)], didx)
        cps = [
            pltpu.async_copy(x_hbm.at[sidx.at[j]],
                             rows.at[pl.ds(j * 128, 128)], gsem)
            for j in range(2)
        ]
        for cp in cps:
            cp.wait()
        for j in range(2):
            pltpu.sync_copy(rows.at[pl.ds(j * 128, 128)],
                            acc.at[didx.at[j]], add=True)
        return 0

    lax.fori_loop(0, rows_per_tile // 2, chunk, 0, unroll=False)
    plsc.subcore_barrier()
    pltpu.sync_copy(acc.at[pl.ds(s * SR, SR)],
                    out_hbm.at[c, pl.ds(s * SR, SR)])


@functools.partial(
    pl.kernel,
    out_type=[jax.ShapeDtypeStruct((4, 2, G, 32), jnp.float32),
              jax.ShapeDtypeStruct((G, 8), jnp.float32)],
    mesh=_mesh,
    scratch_types=[
        pltpu.VMEM((25, 128), jnp.int32),
        pltpu.VMEM((128, 32), jnp.float32),
        pltpu.VMEM((128, 8), jnp.float32),
        pltpu.VMEM_SHARED((GP, 32), jnp.float32),
        pltpu.VMEM_SHARED((GP, 32), jnp.float32),
        pltpu.VMEM_SHARED((GP, 32), jnp.float32),
        pltpu.VMEM_SHARED((GP, 32), jnp.float32),
        pltpu.VMEM_SHARED((GP, 8), jnp.float32),
    ],
    compiler_params=pltpu.CompilerParams(use_tc_tiling_on_sc=False),
)
def _sc_pool(h1, h2, h3, h4, batch_hbm, ones_hbm, z32_hbm, z8_hbm,
             out_pool, out_cnt, bidx, hrow, onesv, a1, a2, a3, a4, ac):
    c = lax.axis_index("c")
    s = lax.axis_index("s")
    accs = [a1, a2, a3, a4]
    hs = [h1, h2, h3, h4]

    @pl.when(s == 0)
    def _init():
        for a in accs:
            pltpu.sync_copy(z32_hbm, a)
        pltpu.sync_copy(z8_hbm, ac)

    pltpu.sync_copy(ones_hbm, onesv)
    pltpu.sync_copy(batch_hbm.at[pl.ds(s * 25, 25)], bidx)
    plsc.subcore_barrier()

    def body(j, _):
        node_base = (s * 25 + j) * 128
        for h, a in zip(hs, accs):
            pltpu.sync_copy(h.at[c, pl.ds(node_base, 128)], hrow)
            pltpu.sync_copy(hrow, a.at[bidx.at[j]], add=True)

        @pl.when(c == 0)
        def _cnt():
            pltpu.sync_copy(onesv, ac.at[bidx.at[j]], add=True)

        return 0

    lax.fori_loop(0, 25, body, 0, unroll=False)
    plsc.subcore_barrier()

    for t, a in enumerate(accs):
        @pl.when(s == t)
        def _wb(a=a, t=t):
            pltpu.sync_copy(a.at[pl.ds(0, G)], out_pool.at[t, c])

    @pl.when((c == 0) & (s == 4))
    def _wbc():
        pltpu.sync_copy(ac.at[pl.ds(0, G)], out_cnt)


NC = 16
CR = NP // NC


def _gelu(z):
    return 0.5 * z * (1.0 + lax.erf(z * (1.0 / jnp.sqrt(2.0).astype(z.dtype))))


def _row_mask(k):
    gi = k * CR + lax.broadcasted_iota(jnp.int32, (CR, 1), 0)
    return (gi < N).astype(jnp.float32)


def _layer_pipeline(load_u, store, w1_ref, b1_ref, g1_ref, bb1_ref, w2_ref,
                    b2_ref, g2_ref, bb2_ref, zbuf):
    def pass1(k, ssum):
        z = jnp.dot(load_u(k), w1_ref[...],
                    preferred_element_type=jnp.float32) + b1_ref[...]
        zbuf[pl.ds(k * CR, CR)] = z
        return ssum + jnp.sum(z * _row_mask(k), 0, keepdims=True)

    def var_of(m, k, vsum):
        d = (zbuf[pl.ds(k * CR, CR)] - m) * _row_mask(k)
        return vsum + jnp.sum(d * d, 0, keepdims=True)

    zero = jnp.zeros((1, HID), jnp.float32)
    m = lax.fori_loop(0, NC, pass1, zero) / N
    v = lax.fori_loop(0, NC, functools.partial(var_of, m), zero) / N
    sc = g1_ref[...] / jnp.sqrt(v + 1e-5)

    def pass2(k, ssum):
        z = _gelu((zbuf[pl.ds(k * CR, CR)] - m) * sc + bb1_ref[...])
        z = jnp.dot(z, w2_ref[...],
                    preferred_element_type=jnp.float32) + b2_ref[...]
        zbuf[pl.ds(k * CR, CR)] = z
        return ssum + jnp.sum(z * _row_mask(k), 0, keepdims=True)

    m2 = lax.fori_loop(0, NC, pass2, zero) / N
    v2 = lax.fori_loop(0, NC, functools.partial(var_of, m2), zero) / N
    sc2 = g2_ref[...] / jnp.sqrt(v2 + 1e-5)

    def pass3(k, _):
        hn = jnp.maximum((zbuf[pl.ds(k * CR, CR)] - m2) * sc2 + bb2_ref[...], 0.0)
        store(k, hn)
        return 0

    lax.fori_loop(0, NC, pass3, 0)


def _tc_layer_mid_body(h_ref, agg_ref, eps_ref, w1_ref, b1_ref, g1_ref,
                       bb1_ref, w2_ref, b2_ref, g2_ref, bb2_ref, out_ref,
                       zbuf, hch, ach, och):
    def load_u(k):
        for p in range(2):
            pltpu.sync_copy(h_ref.at[p, pl.ds(k * CR, CR)], hch.at[p])
            pltpu.sync_copy(agg_ref.at[p, pl.ds(k * CR, CR)], ach.at[p])
        h = jnp.concatenate([hch[0], hch[1]], axis=-1)
        agg = jnp.concatenate([ach[0], ach[1]], axis=-1)
        return (1.0 + eps_ref[0, 0]) * h + agg

    def store(k, hn):
        och[0] = hn[:, :32]
        och[1] = hn[:, 32:]
        for p in range(2):
            pltpu.sync_copy(och.at[p], out_ref.at[p, pl.ds(k * CR, CR)])

    _layer_pipeline(load_u, store, w1_ref, b1_ref, g1_ref, bb1_ref, w2_ref,
                    b2_ref, g2_ref, bb2_ref, zbuf)


def _tc_layer0_body(x_ref, agg_ref, eps_ref, w1_ref, b1_ref, g1_ref,
                    bb1_ref, w2_ref, b2_ref, g2_ref, bb2_ref, out_ref,
                    zbuf, xch, ach, och):
    def load_u(k):
        pltpu.sync_copy(x_ref.at[pl.ds(k * CR, CR)], xch)
        for p in range(2):
            pltpu.sync_copy(agg_ref.at[p, pl.ds(k * CR, CR)], ach.at[p])
        agg = ach[0] + ach[1]
        return (1.0 + eps_ref[0, 0]) * xch[...] + agg

    def store(k, hn):
        och[0] = hn[:, :32]
        och[1] = hn[:, 32:]
        for p in range(2):
            pltpu.sync_copy(och.at[p], out_ref.at[p, pl.ds(k * CR, CR)])

    _layer_pipeline(load_u, store, w1_ref, b1_ref, g1_ref, bb1_ref, w2_ref,
                    b2_ref, g2_ref, bb2_ref, zbuf)


def _tc_head_body(pool_ref, cnt_ref, w1_ref, b1_ref, w2_ref, b2_ref, out_ref):
    cnt = jnp.maximum(cnt_ref[:, 0:1], 1.0)
    parts = [pool_ref[l, c] for l in range(4) for c in range(2)]
    emb = jnp.concatenate(parts, axis=-1) / cnt
    z = _gelu(jnp.dot(emb, w1_ref[...], preferred_element_type=jnp.float32)
              + b1_ref[...])
    out_ref[...] = (jnp.dot(z, w2_ref[...], preferred_element_type=jnp.float32)
                    + b2_ref[...])


_any = pl.BlockSpec(memory_space=pl.ANY)
_vm = pl.BlockSpec(memory_space=pltpu.VMEM)
_wspecs = [_vm] * 10

_tc_layer_mid = pl.pallas_call(
    _tc_layer_mid_body,
    out_shape=jax.ShapeDtypeStruct((2, NP, 32), jnp.float32),
    in_specs=[_any, _any] + [_vm] * 9,
    out_specs=_any,
    scratch_shapes=[pltpu.VMEM((NP, HID), jnp.float32),
                    pltpu.VMEM((2, CR, 32), jnp.float32),
                    pltpu.VMEM((2, CR, 32), jnp.float32),
                    pltpu.VMEM((2, CR, 32), jnp.float32)],
)

_tc_layer0 = pl.pallas_call(
    _tc_layer0_body,
    out_shape=jax.ShapeDtypeStruct((2, NP, 32), jnp.float32),
    in_specs=[_any, _any] + [_vm] * 9,
    out_specs=_any,
    scratch_shapes=[pltpu.VMEM((NP, HID), jnp.float32),
                    pltpu.VMEM((CR, 8), jnp.float32),
                    pltpu.VMEM((2, CR, 8), jnp.float32),
                    pltpu.VMEM((2, CR, 32), jnp.float32)],
)

_tc_head = pl.pallas_call(
    _tc_head_body,
    out_shape=jax.ShapeDtypeStruct((G, 1), jnp.float32),
)


def kernel(x, edge_index, batch, params):
    f32 = jnp.float32
    x8 = jnp.zeros((NP, 8), f32).at[:N, :6].set(x)
    src = jnp.zeros((ER * 128,), jnp.int32).at[:E].set(edge_index[0])
    src = src.reshape(ER, 128)
    dst = jnp.full((ER * 128,), N, jnp.int32).at[:E].set(edge_index[1])
    dst = dst.reshape(ER, 128)
    batchp = jnp.full((BR * 128,), G, jnp.int32).at[:N].set(batch)
    batchp = batchp.reshape(BR, 128)

    z32 = jnp.zeros((SR, 32), f32)
    z8 = jnp.zeros((SR, 8), f32)
    zg32 = jnp.zeros((GP, 32), f32)
    zg8 = jnp.zeros((GP, 8), f32)
    ones = jnp.ones((128, 8), f32)

    agg0 = _sc_agg8(x8, src, dst, z8)
    w1_0 = jnp.zeros((8, HID), f32).at[:6].set(params['c0_W1'])
    h = _tc_layer0(
        x8, agg0, params['c0_eps'].reshape(1, 1), w1_0,
        params['c0_b1'].reshape(1, HID), params['c0_bn1_g'].reshape(1, HID),
        params['c0_bn1_b'].reshape(1, HID), params['c0_W2'],
        params['c0_b2'].reshape(1, HID), params['bn0_g'].reshape(1, HID),
        params['bn0_b'].reshape(1, HID))
    hs = [h]
    for i in range(1, 4):
        agg = _sc_agg64(hs[-1], src, dst, z32)
        h = _tc_layer_mid(
            hs[-1], agg, params[f'c{i}_eps'].reshape(1, 1), params[f'c{i}_W1'],
            params[f'c{i}_b1'].reshape(1, HID),
            params[f'c{i}_bn1_g'].reshape(1, HID),
            params[f'c{i}_bn1_b'].reshape(1, HID), params[f'c{i}_W2'],
            params[f'c{i}_b2'].reshape(1, HID),
            params[f'bn{i}_g'].reshape(1, HID),
            params[f'bn{i}_b'].reshape(1, HID))
        hs.append(h)

    pooled, cnt = _sc_pool(hs[0], hs[1], hs[2], hs[3], batchp, ones, zg32, zg8)
    out = _tc_head(pooled, cnt, params['h_W1'],
                   params['h_b1'].reshape(1, HID), params['h_W2'],
                   params['h_b2'].reshape(1, 1))
    return out[:, 0]
